# Initial kernel scaffold; baseline (speedup 1.0000x reference)
import functools

import jax
import jax.numpy as jnp
from jax import lax
from jax.experimental import pallas as pl
from jax.experimental.pallas import tpu as pltpu
from jax.experimental.pallas import tpu_sc as plsc

N = 10000
E = 320000
D = 128
C = 40
CP = 64
NC, NS = 2, 16
NT = NC * NS
CHUNK = 128
NCH = 79
EPT = NCH * CHUNK
TOT = NT * EPT
NROWS = 10240
RPT = NROWS // NS
BR = 256
GRID = NROWS // BR

_MESH = plsc.VectorSubcoreMesh(core_axis_name="c", subcore_axis_name="s")


def _zero_vmem(buf, n_elems):
    z = jnp.zeros((16,), jnp.float32)

    def body(i, _):
        buf[pl.ds(i * 16, 16)] = z
        return 0

    lax.fori_loop(0, n_elems // 16, body, 0)


def _deg_body(dst_hbm, degp_hbm, didx, ones_v, zbuf, deg_sh):
    c = lax.axis_index("c")
    s = lax.axis_index("s")
    wid = s * NC + c

    _zero_vmem(zbuf, RPT)

    def fill_ones(i, _):
        ones_v[pl.ds(i * 16, 16)] = jnp.ones((16,), jnp.float32)
        return 0

    lax.fori_loop(0, CHUNK // 16, fill_ones, 0)

    base = s * RPT
    pltpu.sync_copy(zbuf, deg_sh.at[pl.ds(base, RPT)])
    pltpu.sync_copy(dst_hbm.at[wid], didx)
    plsc.subcore_barrier()

    def body(j, _):
        pltpu.sync_copy(ones_v, deg_sh.at[didx.at[j]], add=True)
        return 0

    lax.fori_loop(0, NCH, body, 0)
    plsc.subcore_barrier()
    pltpu.sync_copy(deg_sh.at[pl.ds(base, RPT)], degp_hbm.at[c, pl.ds(base, RPT)])


_deg_kernel = pl.kernel(
    _deg_body,
    out_type=jax.ShapeDtypeStruct((NC, NROWS), jnp.float32),
    mesh=_MESH,
    scratch_types=[
        pltpu.VMEM((NCH, CHUNK), jnp.int32),
        pltpu.VMEM((CHUNK,), jnp.float32),
        pltpu.VMEM((RPT,), jnp.float32),
        pltpu.VMEM_SHARED((NROWS,), jnp.float32),
    ],
)


def _hop_body(width, p_hbm, src_hbm, dst_hbm, accp_hbm, sidx, didx, rowbuf,
              acc_sh, sem):
    c = lax.axis_index("c")
    s = lax.axis_index("s")
    wid = s * NC + c

    _zero_vmem(rowbuf, CHUNK * width)
    base = s * RPT
    for t in range(RPT // CHUNK):
        pltpu.sync_copy(rowbuf, acc_sh.at[pl.ds(base + t * CHUNK, CHUNK)])

    pltpu.sync_copy(src_hbm.at[wid], sidx)
    pltpu.sync_copy(dst_hbm.at[wid], didx)
    plsc.subcore_barrier()

    def body(j, _):
        pltpu.async_copy(p_hbm.at[sidx.at[j]], rowbuf, sem).wait()
        pltpu.sync_copy(rowbuf, acc_sh.at[didx.at[j]], add=True)
        return 0

    lax.fori_loop(0, NCH, body, 0)
    plsc.subcore_barrier()
    for t in range(RPT // CHUNK):
        r = base + t * CHUNK
        pltpu.sync_copy(acc_sh.at[pl.ds(r, CHUNK)], accp_hbm.at[c, pl.ds(r, CHUNK)])


def _make_hop(width):
    return pl.kernel(
        functools.partial(_hop_body, width),
        out_type=jax.ShapeDtypeStruct((NC, NROWS, width), jnp.float32),
        mesh=_MESH,
        scratch_types=[
            pltpu.VMEM((NCH, CHUNK), jnp.int32),
            pltpu.VMEM((NCH, CHUNK), jnp.int32),
            pltpu.VMEM((CHUNK, width), jnp.float32),
            pltpu.VMEM_SHARED((NROWS, width), jnp.float32),
            pltpu.SemaphoreType.DMA,
        ],
    )


_hop128 = _make_hop(D)
_hop64 = _make_hop(CP)


def _row_spec(width):
    return pl.BlockSpec((BR, width), lambda i: (i, 0))


def _col_spec(width, kcol):
    return pl.BlockSpec((BR, width), lambda i: (i, kcol))


def _prep_kernel(d0, d1, x, dis_ref, p0_ref):
    deg = d0[...] + d1[...]
    dis = jnp.where(deg > 0, lax.rsqrt(deg), 0.0)
    dis_ref[...] = dis
    p0_ref[...] = dis * x[...]


def _prep_call(d0, d1, x_pad):
    return pl.pallas_call(
        _prep_kernel,
        grid=(GRID,),
        in_specs=[_row_spec(1), _row_spec(1), _row_spec(D)],
        out_specs=[_row_spec(1), _row_spec(D)],
        out_shape=[
            jax.ShapeDtypeStruct((NROWS, 1), jnp.float32),
            jax.ShapeDtypeStruct((NROWS, D), jnp.float32),
        ],
    )(d0, d1, x_pad)


def _comb1_kernel(a0, a1, dis, p_ref, s_ref):
    ssum = a0[...] + a1[...]
    d = dis[...]
    s_ref[...] = ssum
    p_ref[...] = d * d * ssum


def _comb1_call(a0, a1, dis):
    return pl.pallas_call(
        _comb1_kernel,
        grid=(GRID,),
        in_specs=[_row_spec(D), _row_spec(D), _row_spec(1)],
        out_specs=[_row_spec(D), _row_spec(D)],
        out_shape=[
            jax.ShapeDtypeStruct((NROWS, D), jnp.float32),
            jax.ShapeDtypeStruct((NROWS, D), jnp.float32),
        ],
    )(a0, a1, dis)


def _l1fin_kernel(x, s1, s2, a30, a31, dis, W1, b1, W2p, zcat_ref, p3_ref):
    d = dis[...]
    h1 = d * s1[...]
    h2 = d * s2[...]
    h3 = d * (a30[...] + a31[...])
    o = jnp.dot(x[...], W1[0], preferred_element_type=jnp.float32)
    o = o + jnp.dot(h1, W1[1], preferred_element_type=jnp.float32)
    o = o + jnp.dot(h2, W1[2], preferred_element_type=jnp.float32)
    o = o + jnp.dot(h3, W1[3], preferred_element_type=jnp.float32)
    o = o + jnp.sum(b1[...], axis=0)
    h = jnp.maximum(o, 0.0)
    zs = [jnp.dot(h, W2p[k], preferred_element_type=jnp.float32)
          for k in range(4)]
    zcat_ref[...] = jnp.concatenate(zs, axis=1)
    p3_ref[...] = d * zs[3]


def _l1fin_call(x_pad, s1, s2, a30, a31, dis, W1, b1, W2p):
    full3 = lambda shape: pl.BlockSpec(shape, lambda i: (0, 0, 0))
    full2 = lambda shape: pl.BlockSpec(shape, lambda i: (0, 0))
    return pl.pallas_call(
        _l1fin_kernel,
        grid=(GRID,),
        in_specs=[
            _row_spec(D), _row_spec(D), _row_spec(D), _row_spec(D),
            _row_spec(D), _row_spec(1),
            full3((4, D, D)), full2((4, D)), full3((4, D, CP)),
        ],
        out_specs=[_row_spec(4 * CP), _row_spec(CP)],
        out_shape=[
            jax.ShapeDtypeStruct((NROWS, 4 * CP), jnp.float32),
            jax.ShapeDtypeStruct((NROWS, CP), jnp.float32),
        ],
    )(x_pad, s1, s2, a30, a31, dis, W1, b1, W2p)


def _comb2_kernel(z, a0, a1, dis, p_ref):
    d = dis[...]
    p_ref[...] = d * (z[...] + d * (a0[...] + a1[...]))


def _comb2_call(zcat, a0, a1, dis, kcol):
    return pl.pallas_call(
        _comb2_kernel,
        grid=(GRID,),
        in_specs=[_col_spec(CP, kcol), _row_spec(CP), _row_spec(CP),
                  _row_spec(1)],
        out_specs=_row_spec(CP),
        out_shape=jax.ShapeDtypeStruct((NROWS, CP), jnp.float32),
    )(zcat, a0, a1, dis)


def _fin_kernel(z0, a0, a1, dis, b2p, out_ref):
    d = dis[...]
    v = z0[...] + d * (a0[...] + a1[...]) + jnp.sum(b2p[...], axis=0)
    col = lax.broadcasted_iota(jnp.int32, (BR, CP), 1)
    valid = col < C
    vm = jnp.where(valid, v, jnp.float32(-1e30))
    m = jnp.max(vm, axis=1, keepdims=True)
    ex = jnp.where(valid, jnp.exp(vm - m), 0.0)
    lse = jnp.log(jnp.sum(ex, axis=1, keepdims=True))
    out_ref[...] = vm - m - lse


def _fin_call(zcat, a0, a1, dis, b2p):
    return pl.pallas_call(
        _fin_kernel,
        grid=(GRID,),
        in_specs=[_col_spec(CP, 0), _row_spec(CP), _row_spec(CP),
                  _row_spec(1), pl.BlockSpec((4, CP), lambda i: (0, 0))],
        out_specs=_row_spec(CP),
        out_shape=jax.ShapeDtypeStruct((NROWS, CP), jnp.float32),
    )(zcat, a0, a1, dis, b2p)


def kernel(x, edge_index, W1, b1, W2, b2):
    src = edge_index[0].astype(jnp.int32)
    dst = edge_index[1].astype(jnp.int32)
    fill = jnp.arange(TOT - E, dtype=jnp.int32)
    src3 = jnp.concatenate([src, fill % N]).reshape(NT, NCH, CHUNK)
    dst3 = jnp.concatenate([dst, N + fill % (NROWS - N)]).reshape(NT, NCH, CHUNK)
    x_pad = jnp.zeros((NROWS, D), jnp.float32).at[:N].set(x)
    W2p = jnp.zeros((4, D, CP), jnp.float32).at[:, :, :C].set(W2)
    b2p = jnp.zeros((4, CP), jnp.float32).at[:, :C].set(b2)

    degp = _deg_kernel(dst3)
    d0 = degp[0].reshape(NROWS, 1)
    d1 = degp[1].reshape(NROWS, 1)
    dis, p0 = _prep_call(d0, d1, x_pad)

    a1 = _hop128(p0, src3, dst3)
    p1, s1 = _comb1_call(a1[0], a1[1], dis)
    a2 = _hop128(p1, src3, dst3)
    p2, s2 = _comb1_call(a2[0], a2[1], dis)
    a3 = _hop128(p2, src3, dst3)
    zcat, p3 = _l1fin_call(x_pad, s1, s2, a3[0], a3[1], dis, W1, b1, W2p)

    g3 = _hop64(p3, src3, dst3)
    q2 = _comb2_call(zcat, g3[0], g3[1], dis, 2)
    g2 = _hop64(q2, src3, dst3)
    q1 = _comb2_call(zcat, g2[0], g2[1], dis, 1)
    g1 = _hop64(q1, src3, dst3)
    out = _fin_call(zcat, g1[0], g1[1], dis, b2p)
    return out[:N, :C]


# speedup vs baseline: 10.6302x; 10.6302x over previous
import functools

import jax
import jax.numpy as jnp
from jax import lax
from jax.experimental import pallas as pl
from jax.experimental.pallas import tpu as pltpu
from jax.experimental.pallas import tpu_sc as plsc

N = 10000
E = 320000
D = 128
C = 40
CP = 128
NC, NS = 2, 16
NT = NC * NS
CHUNK = 128
NCH = 79
EPT = NCH * CHUNK
TOT = NT * EPT
NROWS = 10240
RPT = NROWS // NS
BR = 256
GRID = NROWS // BR

_MESH = plsc.VectorSubcoreMesh(core_axis_name="c", subcore_axis_name="s")


def _zero_vmem(buf, n_elems):
    z = jnp.zeros((16,), jnp.float32)

    def body(i, _):
        buf[pl.ds(i * 16, 16)] = z
        return 0

    lax.fori_loop(0, n_elems // 16, body, 0)


def _zero_vmem2d(buf, rows, width):
    z = jnp.zeros((16,), jnp.float32)

    def body(r, _):
        for u in range(width // 16):
            buf[r, pl.ds(u * 16, 16)] = z
        return 0

    lax.fori_loop(0, rows, body, 0)


def _deg_body(dst_hbm, degp_hbm, didx, ones_v, zbuf, deg_sh):
    c = lax.axis_index("c")
    s = lax.axis_index("s")
    wid = s * NC + c

    _zero_vmem(zbuf, RPT)

    def fill_ones(i, _):
        ones_v[pl.ds(i * 16, 16)] = jnp.ones((16,), jnp.float32)
        return 0

    lax.fori_loop(0, CHUNK // 16, fill_ones, 0)

    base = s * RPT
    pltpu.sync_copy(zbuf, deg_sh.at[pl.ds(base, RPT)])
    pltpu.sync_copy(dst_hbm.at[wid], didx)
    plsc.subcore_barrier()

    def body(j, _):
        pltpu.sync_copy(ones_v, deg_sh.at[didx.at[j]], add=True)
        return 0

    lax.fori_loop(0, NCH, body, 0)
    plsc.subcore_barrier()
    pltpu.sync_copy(deg_sh.at[pl.ds(base, RPT)], degp_hbm.at[c, pl.ds(base, RPT)])


_deg_kernel = pl.kernel(
    _deg_body,
    out_type=jax.ShapeDtypeStruct((NC, NROWS), jnp.float32),
    mesh=_MESH,
    scratch_types=[
        pltpu.VMEM((NCH, CHUNK), jnp.int32),
        pltpu.VMEM((CHUNK,), jnp.float32),
        pltpu.VMEM((RPT,), jnp.float32),
        pltpu.VMEM_SHARED((NROWS,), jnp.float32),
    ],
)


def _hop_body(width, p_hbm, src_hbm, dst_hbm, accp_hbm, sidx, didx, rowbuf,
              acc_sh, sem):
    c = lax.axis_index("c")
    s = lax.axis_index("s")
    wid = s * NC + c

    _zero_vmem2d(rowbuf, CHUNK, width)
    base = s * RPT
    for t in range(RPT // CHUNK):
        pltpu.sync_copy(rowbuf, acc_sh.at[pl.ds(base + t * CHUNK, CHUNK)])

    pltpu.sync_copy(src_hbm.at[wid], sidx)
    pltpu.sync_copy(dst_hbm.at[wid], didx)
    plsc.subcore_barrier()

    def body(j, _):
        pltpu.async_copy(p_hbm.at[sidx.at[j]], rowbuf, sem).wait()
        pltpu.sync_copy(rowbuf, acc_sh.at[didx.at[j]], add=True)
        return 0

    lax.fori_loop(0, NCH, body, 0)
    plsc.subcore_barrier()
    for t in range(RPT // CHUNK):
        r = base + t * CHUNK
        pltpu.sync_copy(acc_sh.at[pl.ds(r, CHUNK)], accp_hbm.at[c, pl.ds(r, CHUNK)])


def _make_hop(width):
    return pl.kernel(
        functools.partial(_hop_body, width),
        out_type=jax.ShapeDtypeStruct((NC, NROWS, width), jnp.float32),
        mesh=_MESH,
        scratch_types=[
            pltpu.VMEM((NCH, CHUNK), jnp.int32),
            pltpu.VMEM((NCH, CHUNK), jnp.int32),
            pltpu.VMEM((CHUNK, width), jnp.float32),
            pltpu.VMEM_SHARED((NROWS, width), jnp.float32),
            pltpu.SemaphoreType.DMA,
        ],
    )


_hop128 = _make_hop(D)


def _row_spec(width):
    return pl.BlockSpec((BR, width), lambda i: (i, 0))


def _prep_kernel(d0, d1, x, dis_ref, p0_ref):
    deg = d0[...] + d1[...]
    dis = jnp.where(deg > 0, lax.rsqrt(deg), 0.0)
    dis_ref[...] = dis
    p0_ref[...] = dis * x[...]


def _prep_call(d0, d1, x_pad):
    return pl.pallas_call(
        _prep_kernel,
        grid=(GRID,),
        in_specs=[_row_spec(1), _row_spec(1), _row_spec(D)],
        out_specs=[_row_spec(1), _row_spec(D)],
        out_shape=[
            jax.ShapeDtypeStruct((NROWS, 1), jnp.float32),
            jax.ShapeDtypeStruct((NROWS, D), jnp.float32),
        ],
    )(d0, d1, x_pad)


def _comb1_kernel(a0, a1, dis, p_ref, s_ref):
    ssum = a0[...] + a1[...]
    d = dis[...]
    s_ref[...] = ssum
    p_ref[...] = d * d * ssum


def _comb1_call(a0, a1, dis):
    return pl.pallas_call(
        _comb1_kernel,
        grid=(GRID,),
        in_specs=[_row_spec(D), _row_spec(D), _row_spec(1)],
        out_specs=[_row_spec(D), _row_spec(D)],
        out_shape=[
            jax.ShapeDtypeStruct((NROWS, D), jnp.float32),
            jax.ShapeDtypeStruct((NROWS, D), jnp.float32),
        ],
    )(a0, a1, dis)


def _l1fin_kernel(x, s1, s2, a30, a31, dis, W1, b1, W2p,
                  z0_ref, z1_ref, z2_ref, p3_ref):
    d = dis[...]
    h1 = d * s1[...]
    h2 = d * s2[...]
    h3 = d * (a30[...] + a31[...])
    o = jnp.dot(x[...], W1[0], preferred_element_type=jnp.float32)
    o = o + jnp.dot(h1, W1[1], preferred_element_type=jnp.float32)
    o = o + jnp.dot(h2, W1[2], preferred_element_type=jnp.float32)
    o = o + jnp.dot(h3, W1[3], preferred_element_type=jnp.float32)
    o = o + jnp.sum(b1[...], axis=0)
    h = jnp.maximum(o, 0.0)
    zs = [jnp.dot(h, W2p[k], preferred_element_type=jnp.float32)
          for k in range(4)]
    z0_ref[...] = zs[0]
    z1_ref[...] = zs[1]
    z2_ref[...] = zs[2]
    p3_ref[...] = d * zs[3]


def _l1fin_call(x_pad, s1, s2, a30, a31, dis, W1, b1, W2p):
    full3 = lambda shape: pl.BlockSpec(shape, lambda i: (0, 0, 0))
    full2 = lambda shape: pl.BlockSpec(shape, lambda i: (0, 0))
    return pl.pallas_call(
        _l1fin_kernel,
        grid=(GRID,),
        in_specs=[
            _row_spec(D), _row_spec(D), _row_spec(D), _row_spec(D),
            _row_spec(D), _row_spec(1),
            full3((4, D, D)), full2((4, D)), full3((4, D, CP)),
        ],
        out_specs=[_row_spec(CP)] * 4,
        out_shape=[jax.ShapeDtypeStruct((NROWS, CP), jnp.float32)] * 4,
    )(x_pad, s1, s2, a30, a31, dis, W1, b1, W2p)


def _comb2_kernel(z, a0, a1, dis, p_ref):
    d = dis[...]
    p_ref[...] = d * (z[...] + d * (a0[...] + a1[...]))


def _comb2_call(z, a0, a1, dis):
    return pl.pallas_call(
        _comb2_kernel,
        grid=(GRID,),
        in_specs=[_row_spec(CP), _row_spec(CP), _row_spec(CP),
                  _row_spec(1)],
        out_specs=_row_spec(CP),
        out_shape=jax.ShapeDtypeStruct((NROWS, CP), jnp.float32),
    )(z, a0, a1, dis)


def _fin_kernel(z0, a0, a1, dis, b2p, out_ref):
    d = dis[...]
    v = z0[...] + d * (a0[...] + a1[...]) + jnp.sum(b2p[...], axis=0)
    col = lax.broadcasted_iota(jnp.int32, (BR, CP), 1)
    valid = col < C
    vm = jnp.where(valid, v, jnp.float32(-1e30))
    m = jnp.max(vm, axis=1, keepdims=True)
    ex = jnp.where(valid, jnp.exp(vm - m), 0.0)
    lse = jnp.log(jnp.sum(ex, axis=1, keepdims=True))
    out_ref[...] = vm - m - lse


def _fin_call(z0, a0, a1, dis, b2p):
    return pl.pallas_call(
        _fin_kernel,
        grid=(GRID,),
        in_specs=[_row_spec(CP), _row_spec(CP), _row_spec(CP),
                  _row_spec(1), pl.BlockSpec((4, CP), lambda i: (0, 0))],
        out_specs=_row_spec(CP),
        out_shape=jax.ShapeDtypeStruct((NROWS, CP), jnp.float32),
    )(z0, a0, a1, dis, b2p)


def kernel(x, edge_index, W1, b1, W2, b2):
    src = edge_index[0].astype(jnp.int32)
    dst = edge_index[1].astype(jnp.int32)
    fill = jnp.arange(TOT - E, dtype=jnp.int32)
    src3 = jnp.concatenate([src, fill % N]).reshape(NT, NCH, CHUNK)
    dst3 = jnp.concatenate([dst, N + fill % (NROWS - N)]).reshape(NT, NCH, CHUNK)
    x_pad = jnp.zeros((NROWS, D), jnp.float32).at[:N].set(x)
    W2p = jnp.zeros((4, D, CP), jnp.float32).at[:, :, :C].set(W2)
    b2p = jnp.zeros((4, CP), jnp.float32).at[:, :C].set(b2)

    degp = _deg_kernel(dst3)
    d0 = degp[0].reshape(NROWS, 1)
    d1 = degp[1].reshape(NROWS, 1)
    dis, p0 = _prep_call(d0, d1, x_pad)

    a1 = _hop128(p0, src3, dst3)
    p1, s1 = _comb1_call(a1[0], a1[1], dis)
    a2 = _hop128(p1, src3, dst3)
    p2, s2 = _comb1_call(a2[0], a2[1], dis)
    a3 = _hop128(p2, src3, dst3)
    z0, z1, z2, p3 = _l1fin_call(x_pad, s1, s2, a3[0], a3[1], dis,
                                 W1, b1, W2p)

    g3 = _hop128(p3, src3, dst3)
    q2 = _comb2_call(z2, g3[0], g3[1], dis)
    g2 = _hop128(q2, src3, dst3)
    q1 = _comb2_call(z1, g2[0], g2[1], dis)
    g1 = _hop128(q1, src3, dst3)
    out = _fin_call(z0, g1[0], g1[1], dis, b2p)
    return out[:N, :C]


# speedup vs baseline: 15.2921x; 1.4385x over previous
import functools

import jax
import jax.numpy as jnp
from jax import lax
from jax.experimental import pallas as pl
from jax.experimental.pallas import tpu as pltpu
from jax.experimental.pallas import tpu_sc as plsc

N = 10000
E = 320000
D = 128
C = 40
CP = 128
NC, NS = 2, 16
NT = NC * NS
CHUNK = 128
NCH = 80
NBUF = 2
NIX = 2 * NBUF
EPT = NCH * CHUNK
TOT = NT * EPT
NROWS = 10240
RPT = NROWS // NS
BR = 256
GRID = NROWS // BR

_MESH = plsc.VectorSubcoreMesh(core_axis_name="c", subcore_axis_name="s")


def _zero_vmem(buf, n_elems):
    z = jnp.zeros((16,), jnp.float32)

    def body(i, _):
        buf[pl.ds(i * 16, 16)] = z
        return 0

    lax.fori_loop(0, n_elems // 16, body, 0)


def _zero_vmem2d(buf, rows, width):
    z = jnp.zeros((16,), jnp.float32)

    def body(r, _):
        for u in range(width // 16):
            buf[r, pl.ds(u * 16, 16)] = z
        return 0

    lax.fori_loop(0, rows, body, 0)


def _deg_body(dst_hbm, degp_hbm, didx, ones_v, zbuf, deg_sh):
    c = lax.axis_index("c")
    s = lax.axis_index("s")
    wid = s * NC + c

    _zero_vmem(zbuf, RPT)

    def fill_ones(i, _):
        ones_v[pl.ds(i * 16, 16)] = jnp.ones((16,), jnp.float32)
        return 0

    lax.fori_loop(0, CHUNK // 16, fill_ones, 0)

    base = s * RPT
    pltpu.sync_copy(zbuf, deg_sh.at[pl.ds(base, RPT)])
    pltpu.sync_copy(dst_hbm.at[wid], didx)
    plsc.subcore_barrier()

    def body(j, _):
        pltpu.sync_copy(ones_v, deg_sh.at[didx.at[j]], add=True)
        return 0

    lax.fori_loop(0, NCH, body, 0)
    plsc.subcore_barrier()
    pltpu.sync_copy(deg_sh.at[pl.ds(base, RPT)], degp_hbm.at[c, pl.ds(base, RPT)])


_deg_kernel = pl.kernel(
    _deg_body,
    out_type=jax.ShapeDtypeStruct((NC, NROWS), jnp.float32),
    mesh=_MESH,
    scratch_types=[
        pltpu.VMEM((NCH, CHUNK), jnp.int32),
        pltpu.VMEM((CHUNK,), jnp.float32),
        pltpu.VMEM((RPT,), jnp.float32),
        pltpu.VMEM_SHARED((NROWS,), jnp.float32),
    ],
)


def _hop_body(width, p_hbm, ix_hbm, accp_hbm,
              rb0, rb1, ix0, ix1, ix2, ix3, acc_sh,
              sg0, sg1, si0, si1, si2, si3):
    c = lax.axis_index("c")
    s = lax.axis_index("s")
    wid = s * NC + c
    rbs = (rb0, rb1)
    sgs = (sg0, sg1)
    ixq = (ix0, ix1, ix2, ix3)
    siq = (si0, si1, si2, si3)

    _zero_vmem2d(rb0, CHUNK, width)
    base = s * RPT
    for t in range(RPT // CHUNK):
        pltpu.sync_copy(rb0, acc_sh.at[pl.ds(base + t * CHUNK, CHUNK)])
    plsc.subcore_barrier()

    for q in range(NIX):
        pltpu.async_copy(ix_hbm.at[wid, q], ixq[q], siq[q])
    for b in range(NBUF):
        pltpu.make_async_copy(ix_hbm.at[wid, 0], ixq[b], siq[b]).wait()
        pltpu.async_copy(p_hbm.at[ixq[b].at[0]], rbs[b], sgs[b])

    def body(g, _):
        jb = g * NIX
        for q in range(NIX):
            j = jb + q
            b = q % NBUF
            q2 = (q + NBUF) % NIX
            pltpu.make_async_copy(p_hbm.at[ixq[q].at[0]], rbs[b],
                                  sgs[b]).wait()
            pltpu.sync_copy(rbs[b], acc_sh.at[ixq[q].at[1]], add=True)
            pltpu.async_copy(ix_hbm.at[wid, j + NIX], ixq[q], siq[q])
            pltpu.make_async_copy(ix_hbm.at[wid, 0], ixq[q2], siq[q2]).wait()
            pltpu.async_copy(p_hbm.at[ixq[q2].at[0]], rbs[b], sgs[b])
        return 0

    lax.fori_loop(0, NCH // NIX - 1, body, 0)

    for q in range(NIX):
        b = q % NBUF
        pltpu.make_async_copy(p_hbm.at[ixq[q].at[0]], rbs[b], sgs[b]).wait()
        pltpu.sync_copy(rbs[b], acc_sh.at[ixq[q].at[1]], add=True)
        if q < NIX - NBUF:
            q2 = q + NBUF
            pltpu.make_async_copy(ix_hbm.at[wid, 0], ixq[q2], siq[q2]).wait()
            pltpu.async_copy(p_hbm.at[ixq[q2].at[0]], rbs[b], sgs[b])

    plsc.subcore_barrier()
    for t in range(RPT // CHUNK):
        r = base + t * CHUNK
        pltpu.sync_copy(acc_sh.at[pl.ds(r, CHUNK)], accp_hbm.at[c, pl.ds(r, CHUNK)])


def _make_hop(width):
    return pl.kernel(
        functools.partial(_hop_body, width),
        out_type=jax.ShapeDtypeStruct((NC, NROWS, width), jnp.float32),
        mesh=_MESH,
        scratch_types=[
            pltpu.VMEM((CHUNK, width), jnp.float32),
            pltpu.VMEM((CHUNK, width), jnp.float32),
            pltpu.VMEM((2, CHUNK), jnp.int32),
            pltpu.VMEM((2, CHUNK), jnp.int32),
            pltpu.VMEM((2, CHUNK), jnp.int32),
            pltpu.VMEM((2, CHUNK), jnp.int32),
            pltpu.VMEM_SHARED((NROWS, width), jnp.float32),
            pltpu.SemaphoreType.DMA,
            pltpu.SemaphoreType.DMA,
            pltpu.SemaphoreType.DMA,
            pltpu.SemaphoreType.DMA,
            pltpu.SemaphoreType.DMA,
            pltpu.SemaphoreType.DMA,
        ],
    )


_hop128 = _make_hop(D)


def _row_spec(width):
    return pl.BlockSpec((BR, width), lambda i: (i, 0))


def _prep_kernel(d0, d1, x, dis_ref, p0_ref):
    deg = d0[...] + d1[...]
    dis = jnp.where(deg > 0, lax.rsqrt(deg), 0.0)
    dis_ref[...] = dis
    p0_ref[...] = dis * x[...]


def _prep_call(d0, d1, x_pad):
    return pl.pallas_call(
        _prep_kernel,
        grid=(GRID,),
        in_specs=[_row_spec(1), _row_spec(1), _row_spec(D)],
        out_specs=[_row_spec(1), _row_spec(D)],
        out_shape=[
            jax.ShapeDtypeStruct((NROWS, 1), jnp.float32),
            jax.ShapeDtypeStruct((NROWS, D), jnp.float32),
        ],
    )(d0, d1, x_pad)


def _comb1_kernel(a0, a1, dis, p_ref, s_ref):
    ssum = a0[...] + a1[...]
    d = dis[...]
    s_ref[...] = ssum
    p_ref[...] = d * d * ssum


def _comb1_call(a0, a1, dis):
    return pl.pallas_call(
        _comb1_kernel,
        grid=(GRID,),
        in_specs=[_row_spec(D), _row_spec(D), _row_spec(1)],
        out_specs=[_row_spec(D), _row_spec(D)],
        out_shape=[
            jax.ShapeDtypeStruct((NROWS, D), jnp.float32),
            jax.ShapeDtypeStruct((NROWS, D), jnp.float32),
        ],
    )(a0, a1, dis)


def _l1fin_kernel(x, s1, s2, a30, a31, dis, W1, b1, W2p,
                  z0_ref, z1_ref, z2_ref, p3_ref):
    d = dis[...]
    h1 = d * s1[...]
    h2 = d * s2[...]
    h3 = d * (a30[...] + a31[...])
    o = jnp.dot(x[...], W1[0], preferred_element_type=jnp.float32)
    o = o + jnp.dot(h1, W1[1], preferred_element_type=jnp.float32)
    o = o + jnp.dot(h2, W1[2], preferred_element_type=jnp.float32)
    o = o + jnp.dot(h3, W1[3], preferred_element_type=jnp.float32)
    o = o + jnp.sum(b1[...], axis=0)
    h = jnp.maximum(o, 0.0)
    zs = [jnp.dot(h, W2p[k], preferred_element_type=jnp.float32)
          for k in range(4)]
    z0_ref[...] = zs[0]
    z1_ref[...] = zs[1]
    z2_ref[...] = zs[2]
    p3_ref[...] = d * zs[3]


def _l1fin_call(x_pad, s1, s2, a30, a31, dis, W1, b1, W2p):
    full3 = lambda shape: pl.BlockSpec(shape, lambda i: (0, 0, 0))
    full2 = lambda shape: pl.BlockSpec(shape, lambda i: (0, 0))
    return pl.pallas_call(
        _l1fin_kernel,
        grid=(GRID,),
        in_specs=[
            _row_spec(D), _row_spec(D), _row_spec(D), _row_spec(D),
            _row_spec(D), _row_spec(1),
            full3((4, D, D)), full2((4, D)), full3((4, D, CP)),
        ],
        out_specs=[_row_spec(CP)] * 4,
        out_shape=[jax.ShapeDtypeStruct((NROWS, CP), jnp.float32)] * 4,
    )(x_pad, s1, s2, a30, a31, dis, W1, b1, W2p)


def _comb2_kernel(z, a0, a1, dis, p_ref):
    d = dis[...]
    p_ref[...] = d * (z[...] + d * (a0[...] + a1[...]))


def _comb2_call(z, a0, a1, dis):
    return pl.pallas_call(
        _comb2_kernel,
        grid=(GRID,),
        in_specs=[_row_spec(CP), _row_spec(CP), _row_spec(CP),
                  _row_spec(1)],
        out_specs=_row_spec(CP),
        out_shape=jax.ShapeDtypeStruct((NROWS, CP), jnp.float32),
    )(z, a0, a1, dis)


def _fin_kernel(z0, a0, a1, dis, b2p, out_ref):
    d = dis[...]
    v = z0[...] + d * (a0[...] + a1[...]) + jnp.sum(b2p[...], axis=0)
    col = lax.broadcasted_iota(jnp.int32, (BR, CP), 1)
    valid = col < C
    vm = jnp.where(valid, v, jnp.float32(-1e30))
    m = jnp.max(vm, axis=1, keepdims=True)
    ex = jnp.where(valid, jnp.exp(vm - m), 0.0)
    lse = jnp.log(jnp.sum(ex, axis=1, keepdims=True))
    out_ref[...] = vm - m - lse


def _fin_call(z0, a0, a1, dis, b2p):
    return pl.pallas_call(
        _fin_kernel,
        grid=(GRID,),
        in_specs=[_row_spec(CP), _row_spec(CP), _row_spec(CP),
                  _row_spec(1), pl.BlockSpec((4, CP), lambda i: (0, 0))],
        out_specs=_row_spec(CP),
        out_shape=jax.ShapeDtypeStruct((NROWS, CP), jnp.float32),
    )(z0, a0, a1, dis, b2p)


def kernel(x, edge_index, W1, b1, W2, b2):
    src = edge_index[0].astype(jnp.int32)
    dst = edge_index[1].astype(jnp.int32)
    fill = jnp.arange(TOT - E, dtype=jnp.int32)
    src3 = jnp.concatenate([src, fill % N]).reshape(NT, NCH, CHUNK)
    dst3 = jnp.concatenate([dst, N + fill % (NROWS - N)]).reshape(NT, NCH, CHUNK)
    ix3 = jnp.stack([src3, dst3], axis=2)
    x_pad = jnp.zeros((NROWS, D), jnp.float32).at[:N].set(x)
    W2p = jnp.zeros((4, D, CP), jnp.float32).at[:, :, :C].set(W2)
    b2p = jnp.zeros((4, CP), jnp.float32).at[:, :C].set(b2)

    degp = _deg_kernel(dst3)
    d0 = degp[0].reshape(NROWS, 1)
    d1 = degp[1].reshape(NROWS, 1)
    dis, p0 = _prep_call(d0, d1, x_pad)

    a1 = _hop128(p0, ix3)
    p1, s1 = _comb1_call(a1[0], a1[1], dis)
    a2 = _hop128(p1, ix3)
    p2, s2 = _comb1_call(a2[0], a2[1], dis)
    a3 = _hop128(p2, ix3)
    z0, z1, z2, p3 = _l1fin_call(x_pad, s1, s2, a3[0], a3[1], dis,
                                 W1, b1, W2p)

    g3 = _hop128(p3, ix3)
    q2 = _comb2_call(z2, g3[0], g3[1], dis)
    g2 = _hop128(q2, ix3)
    q1 = _comb2_call(z1, g2[0], g2[1], dis)
    g1 = _hop128(q1, ix3)
    out = _fin_call(z0, g1[0], g1[1], dis, b2p)
    return out[:N, :C]


# speedup vs baseline: 16.6940x; 1.0917x over previous
import functools

import jax
import jax.numpy as jnp
from jax import lax
from jax.experimental import pallas as pl
from jax.experimental.pallas import tpu as pltpu
from jax.experimental.pallas import tpu_sc as plsc

N = 10000
E = 320000
D = 128
C = 40
CP = 128
NC, NS = 2, 16
NT = NC * NS
CHUNK = 112
NCH = 90
NBUF = 3
NIX = 2 * NBUF
ZB = 80
EPT = NCH * CHUNK
TOT = NT * EPT
NROWS = 10240
RPT = NROWS // NS
BR = 256
GRID = NROWS // BR

_MESH = plsc.VectorSubcoreMesh(core_axis_name="c", subcore_axis_name="s")


def _zero_vmem(buf, n_elems):
    z = jnp.zeros((16,), jnp.float32)

    def body(i, _):
        buf[pl.ds(i * 16, 16)] = z
        return 0

    lax.fori_loop(0, n_elems // 16, body, 0)


def _zero_vmem2d(buf, rows, width):
    z = jnp.zeros((16,), jnp.float32)

    def body(r, _):
        for u in range(width // 16):
            buf[r, pl.ds(u * 16, 16)] = z
        return 0

    lax.fori_loop(0, rows, body, 0)


def _deg_body(dst_hbm, degp_hbm, didx, ones_v, zbuf, deg_sh, dsem):
    c = lax.axis_index("c")
    s = lax.axis_index("s")
    wid = s * NC + c

    _zero_vmem(zbuf, RPT)

    def fill_ones(i, _):
        ones_v[pl.ds(i * 16, 16)] = jnp.ones((16,), jnp.float32)
        return 0

    lax.fori_loop(0, CHUNK // 16, fill_ones, 0)

    base = s * RPT
    pltpu.sync_copy(zbuf, deg_sh.at[pl.ds(base, RPT)])
    pltpu.sync_copy(dst_hbm.at[wid], didx)
    plsc.subcore_barrier()

    def body(j, _):
        pltpu.async_copy(ones_v, deg_sh.at[didx.at[j]], dsem, add=True)
        return 0

    lax.fori_loop(0, NCH, body, 0)

    def drain(j, _):
        pltpu.make_async_copy(ones_v, deg_sh.at[didx.at[j]], dsem).wait()
        return 0

    lax.fori_loop(0, NCH, drain, 0)
    plsc.subcore_barrier()
    pltpu.sync_copy(deg_sh.at[pl.ds(base, RPT)], degp_hbm.at[c, pl.ds(base, RPT)])


_deg_kernel = pl.kernel(
    _deg_body,
    out_type=jax.ShapeDtypeStruct((NC, NROWS), jnp.float32),
    mesh=_MESH,
    scratch_types=[
        pltpu.VMEM((NCH, CHUNK), jnp.int32),
        pltpu.VMEM((CHUNK,), jnp.float32),
        pltpu.VMEM((RPT,), jnp.float32),
        pltpu.VMEM_SHARED((NROWS,), jnp.float32),
        pltpu.SemaphoreType.DMA,
    ],
)


def _hop_body(width, p_hbm, ix_hbm, accp_hbm,
              rb0, rb1, rb2, ix0, ix1, ix2, ix3, ix4, ix5, acc_sh,
              sg0, sg1, sg2, si0, si1, si2, si3, si4, si5):
    c = lax.axis_index("c")
    s = lax.axis_index("s")
    wid = s * NC + c
    rbs = (rb0, rb1, rb2)
    sgs = (sg0, sg1, sg2)
    ixq = (ix0, ix1, ix2, ix3, ix4, ix5)
    siq = (si0, si1, si2, si3, si4, si5)

    _zero_vmem2d(rb0, ZB, width)
    base = s * RPT
    for t in range(RPT // ZB):
        pltpu.sync_copy(rb0.at[pl.ds(0, ZB)],
                        acc_sh.at[pl.ds(base + t * ZB, ZB)])
    plsc.subcore_barrier()

    for q in range(NIX):
        pltpu.async_copy(ix_hbm.at[wid, q], ixq[q], siq[q])
    for b in range(NBUF):
        pltpu.make_async_copy(ix_hbm.at[wid, 0], ixq[b], siq[b]).wait()
        pltpu.async_copy(p_hbm.at[ixq[b].at[0]], rbs[b], sgs[b])

    def body(g, _):
        jb = g * NIX
        for q in range(NIX):
            j = jb + q
            b = q % NBUF
            q2 = (q + NBUF) % NIX
            pltpu.make_async_copy(p_hbm.at[ixq[q].at[0]], rbs[b],
                                  sgs[b]).wait()
            pltpu.sync_copy(rbs[b], acc_sh.at[ixq[q].at[1]], add=True)
            pltpu.async_copy(ix_hbm.at[wid, j + NIX], ixq[q], siq[q])
            pltpu.make_async_copy(ix_hbm.at[wid, 0], ixq[q2], siq[q2]).wait()
            pltpu.async_copy(p_hbm.at[ixq[q2].at[0]], rbs[b], sgs[b])
        return 0

    lax.fori_loop(0, NCH // NIX - 1, body, 0)

    for q in range(NIX):
        b = q % NBUF
        pltpu.make_async_copy(p_hbm.at[ixq[q].at[0]], rbs[b], sgs[b]).wait()
        pltpu.sync_copy(rbs[b], acc_sh.at[ixq[q].at[1]], add=True)
        if q < NIX - NBUF:
            q2 = q + NBUF
            pltpu.make_async_copy(ix_hbm.at[wid, 0], ixq[q2], siq[q2]).wait()
            pltpu.async_copy(p_hbm.at[ixq[q2].at[0]], rbs[b], sgs[b])

    plsc.subcore_barrier()
    for t in range(RPT // ZB):
        r = base + t * ZB
        pltpu.sync_copy(acc_sh.at[pl.ds(r, ZB)], accp_hbm.at[c, pl.ds(r, ZB)])


def _make_hop(width):
    return pl.kernel(
        functools.partial(_hop_body, width),
        out_type=jax.ShapeDtypeStruct((NC, NROWS, width), jnp.float32),
        mesh=_MESH,
        scratch_types=[
            pltpu.VMEM((CHUNK, width), jnp.float32),
            pltpu.VMEM((CHUNK, width), jnp.float32),
            pltpu.VMEM((CHUNK, width), jnp.float32),
            pltpu.VMEM((2, CHUNK), jnp.int32),
            pltpu.VMEM((2, CHUNK), jnp.int32),
            pltpu.VMEM((2, CHUNK), jnp.int32),
            pltpu.VMEM((2, CHUNK), jnp.int32),
            pltpu.VMEM((2, CHUNK), jnp.int32),
            pltpu.VMEM((2, CHUNK), jnp.int32),
            pltpu.VMEM_SHARED((NROWS, width), jnp.float32),
        ] + [pltpu.SemaphoreType.DMA] * 9,
    )


_hop128 = _make_hop(D)


def _row_spec(width):
    return pl.BlockSpec((BR, width), lambda i: (i, 0))


def _prep_kernel(d0, d1, x, dis_ref, p0_ref):
    deg = d0[...] + d1[...]
    dis = jnp.where(deg > 0, lax.rsqrt(deg), 0.0)
    dis_ref[...] = dis
    p0_ref[...] = dis * x[...]


def _prep_call(d0, d1, x_pad):
    return pl.pallas_call(
        _prep_kernel,
        grid=(GRID,),
        in_specs=[_row_spec(1), _row_spec(1), _row_spec(D)],
        out_specs=[_row_spec(1), _row_spec(D)],
        out_shape=[
            jax.ShapeDtypeStruct((NROWS, 1), jnp.float32),
            jax.ShapeDtypeStruct((NROWS, D), jnp.float32),
        ],
    )(d0, d1, x_pad)


def _comb1_kernel(a0, a1, dis, p_ref, s_ref):
    ssum = a0[...] + a1[...]
    d = dis[...]
    s_ref[...] = ssum
    p_ref[...] = d * d * ssum


def _comb1_call(a0, a1, dis):
    return pl.pallas_call(
        _comb1_kernel,
        grid=(GRID,),
        in_specs=[_row_spec(D), _row_spec(D), _row_spec(1)],
        out_specs=[_row_spec(D), _row_spec(D)],
        out_shape=[
            jax.ShapeDtypeStruct((NROWS, D), jnp.float32),
            jax.ShapeDtypeStruct((NROWS, D), jnp.float32),
        ],
    )(a0, a1, dis)


def _l1fin_kernel(x, s1, s2, a30, a31, dis, W1, b1, W2p,
                  z0_ref, z1_ref, z2_ref, p3_ref):
    d = dis[...]
    h1 = d * s1[...]
    h2 = d * s2[...]
    h3 = d * (a30[...] + a31[...])
    o = jnp.dot(x[...], W1[0], preferred_element_type=jnp.float32)
    o = o + jnp.dot(h1, W1[1], preferred_element_type=jnp.float32)
    o = o + jnp.dot(h2, W1[2], preferred_element_type=jnp.float32)
    o = o + jnp.dot(h3, W1[3], preferred_element_type=jnp.float32)
    o = o + jnp.sum(b1[...], axis=0)
    h = jnp.maximum(o, 0.0)
    zs = [jnp.dot(h, W2p[k], preferred_element_type=jnp.float32)
          for k in range(4)]
    z0_ref[...] = zs[0]
    z1_ref[...] = zs[1]
    z2_ref[...] = zs[2]
    p3_ref[...] = d * zs[3]


def _l1fin_call(x_pad, s1, s2, a30, a31, dis, W1, b1, W2p):
    full3 = lambda shape: pl.BlockSpec(shape, lambda i: (0, 0, 0))
    full2 = lambda shape: pl.BlockSpec(shape, lambda i: (0, 0))
    return pl.pallas_call(
        _l1fin_kernel,
        grid=(GRID,),
        in_specs=[
            _row_spec(D), _row_spec(D), _row_spec(D), _row_spec(D),
            _row_spec(D), _row_spec(1),
            full3((4, D, D)), full2((4, D)), full3((4, D, CP)),
        ],
        out_specs=[_row_spec(CP)] * 4,
        out_shape=[jax.ShapeDtypeStruct((NROWS, CP), jnp.float32)] * 4,
    )(x_pad, s1, s2, a30, a31, dis, W1, b1, W2p)


def _comb2_kernel(z, a0, a1, dis, p_ref):
    d = dis[...]
    p_ref[...] = d * (z[...] + d * (a0[...] + a1[...]))


def _comb2_call(z, a0, a1, dis):
    return pl.pallas_call(
        _comb2_kernel,
        grid=(GRID,),
        in_specs=[_row_spec(CP), _row_spec(CP), _row_spec(CP),
                  _row_spec(1)],
        out_specs=_row_spec(CP),
        out_shape=jax.ShapeDtypeStruct((NROWS, CP), jnp.float32),
    )(z, a0, a1, dis)


def _fin_kernel(z0, a0, a1, dis, b2p, out_ref):
    d = dis[...]
    v = z0[...] + d * (a0[...] + a1[...]) + jnp.sum(b2p[...], axis=0)
    col = lax.broadcasted_iota(jnp.int32, (BR, CP), 1)
    valid = col < C
    vm = jnp.where(valid, v, jnp.float32(-1e30))
    m = jnp.max(vm, axis=1, keepdims=True)
    ex = jnp.where(valid, jnp.exp(vm - m), 0.0)
    lse = jnp.log(jnp.sum(ex, axis=1, keepdims=True))
    out_ref[...] = vm - m - lse


def _fin_call(z0, a0, a1, dis, b2p):
    return pl.pallas_call(
        _fin_kernel,
        grid=(GRID,),
        in_specs=[_row_spec(CP), _row_spec(CP), _row_spec(CP),
                  _row_spec(1), pl.BlockSpec((4, CP), lambda i: (0, 0))],
        out_specs=_row_spec(CP),
        out_shape=jax.ShapeDtypeStruct((NROWS, CP), jnp.float32),
    )(z0, a0, a1, dis, b2p)


def kernel(x, edge_index, W1, b1, W2, b2):
    src = edge_index[0].astype(jnp.int32)
    dst = edge_index[1].astype(jnp.int32)
    fill = jnp.arange(TOT - E, dtype=jnp.int32)
    src3 = jnp.concatenate([src, fill % N]).reshape(NT, NCH, CHUNK)
    dst3 = jnp.concatenate([dst, N + fill % (NROWS - N)]).reshape(NT, NCH, CHUNK)
    ix3 = jnp.stack([src3, dst3], axis=2)
    x_pad = jnp.zeros((NROWS, D), jnp.float32).at[:N].set(x)
    W2p = jnp.zeros((4, D, CP), jnp.float32).at[:, :, :C].set(W2)
    b2p = jnp.zeros((4, CP), jnp.float32).at[:, :C].set(b2)

    degp = _deg_kernel(dst3)
    d0 = degp[0].reshape(NROWS, 1)
    d1 = degp[1].reshape(NROWS, 1)
    dis, p0 = _prep_call(d0, d1, x_pad)

    a1 = _hop128(p0, ix3)
    p1, s1 = _comb1_call(a1[0], a1[1], dis)
    a2 = _hop128(p1, ix3)
    p2, s2 = _comb1_call(a2[0], a2[1], dis)
    a3 = _hop128(p2, ix3)
    z0, z1, z2, p3 = _l1fin_call(x_pad, s1, s2, a3[0], a3[1], dis,
                                 W1, b1, W2p)

    g3 = _hop128(p3, ix3)
    q2 = _comb2_call(z2, g3[0], g3[1], dis)
    g2 = _hop128(q2, ix3)
    q1 = _comb2_call(z1, g2[0], g2[1], dis)
    g1 = _hop128(q1, ix3)
    out = _fin_call(z0, g1[0], g1[1], dis, b2p)
    return out[:N, :C]
